# initial kernel scaffold (unmeasured)
import jax
import jax.numpy as jnp
from jax import lax
from jax.experimental import pallas as pl
from jax.experimental.pallas import tpu as pltpu

N_DEV = 32


def kernel(x, w_mat):
    m_per, k = x.shape
    _, n_per = w_mat.shape

    def body(x_ref, w_ref, out_ref, comm_ref, send_sems, recv_sems,
             bf_send, bf_recv, bf_ssem, bf_rsem):
        my_pos = lax.axis_index("i")
        right = lax.rem(my_pos + 1, N_DEV)


        comm_ref[0] = x_ref[...]
        out_ref[pl.ds(my_pos * m_per, m_per), :] = jnp.dot(
            x_ref[...], w_ref[...], preferred_element_type=jnp.float32
        )

        for h in range(N_DEV - 1):
            send_slot = h % 2
            recv_slot = (h + 1) % 2
            rdma = pltpu.make_async_remote_copy(
                src_ref=comm_ref.at[send_slot],
                dst_ref=comm_ref.at[recv_slot],
                send_sem=send_sems.at[send_slot],
                recv_sem=recv_sems.at[recv_slot],
                device_id=(right,),
                device_id_type=pl.DeviceIdType.MESH,
            )
            rdma.start()
            rdma.wait()

            origin = lax.rem(my_pos - (h + 1) + N_DEV, N_DEV)
            out_ref[pl.ds(origin * m_per, m_per), :] = jnp.dot(
                comm_ref[recv_slot], w_ref[...],
                preferred_element_type=jnp.float32,
            )

        amax = jnp.max(jnp.abs(out_ref[...]))
        for s in range(5):
            partner = jnp.bitwise_xor(my_pos, 1 << s)
            bf_send[s] = jnp.full((8, 128), amax, dtype=jnp.float32)
            rdma = pltpu.make_async_remote_copy(
                src_ref=bf_send.at[s],
                dst_ref=bf_recv.at[s],
                send_sem=bf_ssem.at[s],
                recv_sem=bf_rsem.at[s],
                device_id=(partner,),
                device_id_type=pl.DeviceIdType.MESH,
            )
            rdma.start()
            rdma.wait()
            amax = jnp.maximum(amax, bf_recv[s, 0, 0])

        scale = amax / 448.0
        q = jnp.clip(out_ref[...] / scale, -448.0, 448.0)
        out_ref[...] = q.astype(jnp.float8_e4m3fn).astype(jnp.float32) * scale

    return pl.pallas_call(
        body,
        out_shape=jax.ShapeDtypeStruct((N_DEV * m_per, n_per), jnp.float32),
        in_specs=[
            pl.BlockSpec(memory_space=pltpu.VMEM),
            pl.BlockSpec(memory_space=pltpu.VMEM),
        ],
        out_specs=pl.BlockSpec(memory_space=pltpu.VMEM),
        scratch_shapes=[
            pltpu.VMEM((2, m_per, k), jnp.float32),
            pltpu.SemaphoreType.DMA((2,)),
            pltpu.SemaphoreType.DMA((2,)),
            pltpu.VMEM((5, 8, 128), jnp.float32),
            pltpu.VMEM((5, 8, 128), jnp.float32),
            pltpu.SemaphoreType.DMA((5,)),
            pltpu.SemaphoreType.DMA((5,)),
        ],
        compiler_params=pltpu.CompilerParams(collective_id=0),
    )(x, w_mat)


# baseline (device time: 797531 ns/iter reference)
import jax
import jax.numpy as jnp
from jax import lax
from jax.experimental import pallas as pl
from jax.experimental.pallas import tpu as pltpu

N_DEV = 32


def kernel(x, w_mat):
    m_per, k = x.shape
    _, n_per = w_mat.shape

    def body(x_ref, w_ref, out_ref, comm_ref, send_sems, recv_sems,
             bf_send, bf_recv, bf_ssem, bf_rsem):
        my_pos = lax.axis_index("i")
        right = lax.rem(my_pos + 1, N_DEV)


        comm_ref[0] = x_ref[...]
        out_ref[pl.ds(my_pos * m_per, m_per), :] = jnp.dot(
            x_ref[...], w_ref[...], preferred_element_type=jnp.float32
        )

        for h in range(N_DEV - 1):
            send_slot = h % 2
            recv_slot = (h + 1) % 2
            rdma = pltpu.make_async_remote_copy(
                src_ref=comm_ref.at[send_slot],
                dst_ref=comm_ref.at[recv_slot],
                send_sem=send_sems.at[send_slot],
                recv_sem=recv_sems.at[recv_slot],
                device_id=(right,),
                device_id_type=pl.DeviceIdType.MESH,
            )
            rdma.start()
            rdma.wait()

            origin = lax.rem(my_pos - (h + 1) + N_DEV, N_DEV)
            out_ref[pl.ds(origin * m_per, m_per), :] = jnp.dot(
                comm_ref[recv_slot], w_ref[...],
                preferred_element_type=jnp.float32,
            )

        amax = jnp.max(jnp.abs(out_ref[...]))
        for s in range(5):
            partner = jnp.bitwise_xor(my_pos, 1 << s)
            bf_send[s] = jnp.full((8, 128), amax, dtype=jnp.float32)
            rdma = pltpu.make_async_remote_copy(
                src_ref=bf_send.at[s],
                dst_ref=bf_recv.at[s],
                send_sem=bf_ssem.at[s],
                recv_sem=bf_rsem.at[s],
                device_id=(partner,),
                device_id_type=pl.DeviceIdType.MESH,
            )
            rdma.start()
            rdma.wait()
            amax = jnp.maximum(amax, bf_recv[s, 0, 0])

        scale = amax / 448.0
        q = jnp.clip(out_ref[...] / scale, -448.0, 448.0)
        out_ref[...] = q.astype(jnp.float8_e4m3fn).astype(jnp.float32) * scale

    return pl.pallas_call(
        body,
        out_shape=jax.ShapeDtypeStruct((N_DEV * m_per, n_per), jnp.float32),
        in_specs=[
            pl.BlockSpec(memory_space=pltpu.VMEM),
            pl.BlockSpec(memory_space=pltpu.VMEM),
        ],
        out_specs=pl.BlockSpec(memory_space=pltpu.VMEM),
        scratch_shapes=[
            pltpu.VMEM((2, m_per, k), jnp.float32),
            pltpu.SemaphoreType.DMA((2,)),
            pltpu.SemaphoreType.DMA((2,)),
            pltpu.VMEM((5, 8, 128), jnp.float32),
            pltpu.VMEM((5, 8, 128), jnp.float32),
            pltpu.SemaphoreType.DMA((5,)),
            pltpu.SemaphoreType.DMA((5,)),
        ],
    )(x, w_mat)


# device time: 415901 ns/iter; 1.9176x vs baseline; 1.9176x over previous
import jax
import jax.numpy as jnp
import numpy as np
from jax import lax
from jax.experimental import pallas as pl
from jax.experimental.pallas import tpu as pltpu

N_DEV = 32


def _mesh_idx(x, y, z):
    return z * 8 + y * 2 + (x if y % 2 == 0 else 1 - x)


def _ring_mesh_indices():
    p = []
    for z in range(4):
        ys = range(4) if z % 2 == 0 else range(3, -1, -1)
        for y in ys:
            p.append((y, z))
    cycle = [(0, y, z) for (y, z) in p] + [(1, y, z) for (y, z) in reversed(p)]
    return [_mesh_idx(x, y, z) for (x, y, z) in cycle]

_RING2MESH = _ring_mesh_indices()
_MESH2RING = [0] * N_DEV
for _r, _m in enumerate(_RING2MESH):
    _MESH2RING[_m] = _r

N_FWD = 16
N_BWD = 15


def kernel(x, w_mat):
    m_per, k = x.shape
    _, n_per = w_mat.shape

    ring2mesh = jnp.array(_RING2MESH, dtype=jnp.int32)
    mesh2ring = jnp.array(_MESH2RING, dtype=jnp.int32)

    def body(x_ref, w_ref, r2m_ref, m2r_ref, out_ref,
             comm_f, comm_b, ssem_f, rsem_f, ssem_b, rsem_b,
             bf_send, bf_recv, bf_ssem, bf_rsem):
        my_pos = lax.axis_index("i")
        r_me = m2r_ref[my_pos]
        fwd_nbr = r2m_ref[lax.rem(r_me + 1, N_DEV)]
        bwd_nbr = r2m_ref[lax.rem(r_me - 1 + N_DEV, N_DEV)]

        def origin_at(ring_offset):
            return r2m_ref[lax.rem(r_me + ring_offset + 2 * N_DEV, N_DEV)]

        def chunk_gemm(chunk, origin):
            out_ref[pl.ds(origin * m_per, m_per), :] = jnp.dot(
                chunk, w_ref[...], preferred_element_type=jnp.float32
            )

        comm_f[0] = x_ref[...]
        comm_b[0] = x_ref[...]

        def hop(comm, ssem, rsem, s, nbr):
            rdma = pltpu.make_async_remote_copy(
                src_ref=comm.at[s % 2],
                dst_ref=comm.at[(s + 1) % 2],
                send_sem=ssem.at[s % 2],
                recv_sem=rsem.at[(s + 1) % 2],
                device_id=(nbr,),
                device_id_type=pl.DeviceIdType.MESH,
            )
            rdma.start()
            return rdma

        for s in range(N_FWD):
            rf = hop(comm_f, ssem_f, rsem_f, s, fwd_nbr)
            rb = hop(comm_b, ssem_b, rsem_b, s, bwd_nbr) if s < N_BWD else None
            if s == 0:
                chunk_gemm(x_ref[...], my_pos)
            else:
                chunk_gemm(comm_f[s % 2], origin_at(-s))
                if s < N_BWD:
                    chunk_gemm(comm_b[s % 2], origin_at(s))
            rf.wait()
            if rb is not None:
                rb.wait()
        chunk_gemm(comm_f[N_FWD % 2], origin_at(-N_FWD))
        chunk_gemm(comm_b[N_BWD % 2], origin_at(N_BWD))

        amax = jnp.max(jnp.abs(out_ref[...]))
        for s in range(5):
            partner = jnp.bitwise_xor(my_pos, 1 << s)
            bf_send[s] = jnp.full((8, 128), amax, dtype=jnp.float32)
            rdma = pltpu.make_async_remote_copy(
                src_ref=bf_send.at[s],
                dst_ref=bf_recv.at[s],
                send_sem=bf_ssem.at[s],
                recv_sem=bf_rsem.at[s],
                device_id=(partner,),
                device_id_type=pl.DeviceIdType.MESH,
            )
            rdma.start()
            rdma.wait()
            amax = jnp.maximum(amax, bf_recv[s, 0, 0])

        scale = amax / 448.0
        q = jnp.clip(out_ref[...] / scale, -448.0, 448.0)
        out_ref[...] = q.astype(jnp.float8_e4m3fn).astype(jnp.float32) * scale

    return pl.pallas_call(
        body,
        out_shape=jax.ShapeDtypeStruct((N_DEV * m_per, n_per), jnp.float32),
        in_specs=[
            pl.BlockSpec(memory_space=pltpu.VMEM),
            pl.BlockSpec(memory_space=pltpu.VMEM),
            pl.BlockSpec(memory_space=pltpu.SMEM),
            pl.BlockSpec(memory_space=pltpu.SMEM),
        ],
        out_specs=pl.BlockSpec(memory_space=pltpu.VMEM),
        scratch_shapes=[
            pltpu.VMEM((2, m_per, k), jnp.float32),
            pltpu.VMEM((2, m_per, k), jnp.float32),
            pltpu.SemaphoreType.DMA((2,)),
            pltpu.SemaphoreType.DMA((2,)),
            pltpu.SemaphoreType.DMA((2,)),
            pltpu.SemaphoreType.DMA((2,)),
            pltpu.VMEM((5, 8, 128), jnp.float32),
            pltpu.VMEM((5, 8, 128), jnp.float32),
            pltpu.SemaphoreType.DMA((5,)),
            pltpu.SemaphoreType.DMA((5,)),
        ],
    )(x, w_mat, ring2mesh, mesh2ring)


# device time: 390035 ns/iter; 2.0448x vs baseline; 1.0663x over previous
import jax
import jax.numpy as jnp
from jax import lax
from jax.experimental import pallas as pl
from jax.experimental.pallas import tpu as pltpu

N_DEV = 32


def _mesh_idx(x, y, z):
    return z * 8 + y * 2 + (x if y % 2 == 0 else 1 - x)


def _ring_mesh_indices():
    p = []
    for z in range(4):
        ys = range(4) if z % 2 == 0 else range(3, -1, -1)
        for y in ys:
            p.append((y, z))
    cycle = [(0, y, z) for (y, z) in p] + [(1, y, z) for (y, z) in reversed(p)]
    return [_mesh_idx(x, y, z) for (x, y, z) in cycle]

_RING2MESH = _ring_mesh_indices()
_MESH2RING = [0] * N_DEV
for _r, _m in enumerate(_RING2MESH):
    _MESH2RING[_m] = _r

N_FWD = 16
N_BWD = 15
DEPTH = 4


def kernel(x, w_mat):
    m_per, k = x.shape
    _, n_per = w_mat.shape
    H = m_per // 2

    ring2mesh = jnp.array(_RING2MESH, dtype=jnp.int32)
    mesh2ring = jnp.array(_MESH2RING, dtype=jnp.int32)

    def body(x_ref, w_ref, r2m_ref, m2r_ref, out_ref,
             comm_f, comm_b, ssem_f, rsem_f, ssem_b, rsem_b,
             cred_f0, cred_f1, cred_b0, cred_b1,
             bf_send, bf_recv, bf_ssem, bf_rsem):
        my_pos = lax.axis_index("i")
        r_me = m2r_ref[my_pos]
        fwd_nbr = r2m_ref[lax.rem(r_me + 1, N_DEV)]
        bwd_nbr = r2m_ref[lax.rem(r_me - 1 + N_DEV, N_DEV)]
        cred_f = [cred_f0, cred_f1]
        cred_b = [cred_b0, cred_b1]

        def origin_at(ring_offset):
            return r2m_ref[lax.rem(r_me + ring_offset + 2 * N_DEV, N_DEV)]

        def mk(comm, ssem, rsem, j, t, nbr):
            return pltpu.make_async_remote_copy(
                src_ref=comm.at[j % DEPTH, t * H:(t + 1) * H],
                dst_ref=comm.at[(j + 1) % DEPTH, t * H:(t + 1) * H],
                send_sem=ssem.at[j % DEPTH, t],
                recv_sem=rsem.at[(j + 1) % DEPTH, t],
                device_id=(nbr,),
                device_id_type=pl.DeviceIdType.MESH,
            )

        def half_gemm(src, origin, t, amax):
            res = jnp.dot(src, w_ref[...], preferred_element_type=jnp.float32)
            out_ref[pl.ds(origin * m_per + t * H, H), :] = res
            return jnp.maximum(amax, jnp.max(jnp.abs(res)))

        comm_f[0] = x_ref[...]
        comm_b[0] = x_ref[...]
        for t in (0, 1):
            mk(comm_f, ssem_f, rsem_f, 0, t, fwd_nbr).start()
        for t in (0, 1):
            mk(comm_b, ssem_b, rsem_b, 0, t, bwd_nbr).start()

        amax = jnp.float32(0.0)
        for t in (0, 1):
            amax = half_gemm(x_ref[t * H:(t + 1) * H], my_pos, t, amax)

        for s in range(1, N_FWD + 1):
            for t in (0, 1):
                mk(comm_f, ssem_f, rsem_f, s - 1, t, fwd_nbr).wait_recv()
                if s < N_FWD:
                    if s >= DEPTH - 1:
                        pl.semaphore_wait(cred_f[t], 1)
                    mk(comm_f, ssem_f, rsem_f, s, t, fwd_nbr).start()
            if s <= N_BWD:
                for t in (0, 1):
                    mk(comm_b, ssem_b, rsem_b, s - 1, t, bwd_nbr).wait_recv()
                    if s < N_BWD:
                        if s >= DEPTH - 1:
                            pl.semaphore_wait(cred_b[t], 1)
                        mk(comm_b, ssem_b, rsem_b, s, t, bwd_nbr).start()

            o_f = origin_at(-s)
            for t in (0, 1):
                amax = half_gemm(comm_f[s % DEPTH, t * H:(t + 1) * H], o_f, t, amax)
            if s <= N_BWD:
                o_b = origin_at(s)
                for t in (0, 1):
                    amax = half_gemm(comm_b[s % DEPTH, t * H:(t + 1) * H], o_b, t, amax)

            kk = s - 2
            if 0 <= kk <= N_FWD - 1:
                for t in (0, 1):
                    mk(comm_f, ssem_f, rsem_f, kk, t, fwd_nbr).wait_send()
                    if kk <= N_FWD - DEPTH:
                        pl.semaphore_signal(
                            cred_f[t], inc=1,
                            device_id=(bwd_nbr,),
                            device_id_type=pl.DeviceIdType.MESH,
                        )
            if 0 <= kk <= N_BWD - 1:
                for t in (0, 1):
                    mk(comm_b, ssem_b, rsem_b, kk, t, bwd_nbr).wait_send()
                    if kk <= N_BWD - DEPTH:
                        pl.semaphore_signal(
                            cred_b[t], inc=1,
                            device_id=(fwd_nbr,),
                            device_id_type=pl.DeviceIdType.MESH,
                        )

        for t in (0, 1):
            mk(comm_f, ssem_f, rsem_f, N_FWD - 1, t, fwd_nbr).wait_send()

        for s in range(5):
            partner = jnp.bitwise_xor(my_pos, 1 << s)
            bf_send[s] = jnp.full((8, 128), amax, dtype=jnp.float32)
            rdma = pltpu.make_async_remote_copy(
                src_ref=bf_send.at[s],
                dst_ref=bf_recv.at[s],
                send_sem=bf_ssem.at[s],
                recv_sem=bf_rsem.at[s],
                device_id=(partner,),
                device_id_type=pl.DeviceIdType.MESH,
            )
            rdma.start()
            rdma.wait()
            amax = jnp.maximum(amax, bf_recv[s, 0, 0])

        scale = amax / 448.0
        q = jnp.clip(out_ref[...] / scale, -448.0, 448.0)
        out_ref[...] = q.astype(jnp.float8_e4m3fn).astype(jnp.float32) * scale

    return pl.pallas_call(
        body,
        out_shape=jax.ShapeDtypeStruct((N_DEV * m_per, n_per), jnp.float32),
        in_specs=[
            pl.BlockSpec(memory_space=pltpu.VMEM),
            pl.BlockSpec(memory_space=pltpu.VMEM),
            pl.BlockSpec(memory_space=pltpu.SMEM),
            pl.BlockSpec(memory_space=pltpu.SMEM),
        ],
        out_specs=pl.BlockSpec(memory_space=pltpu.VMEM),
        scratch_shapes=[
            pltpu.VMEM((DEPTH, m_per, k), jnp.float32),
            pltpu.VMEM((DEPTH, m_per, k), jnp.float32),
            pltpu.SemaphoreType.DMA((DEPTH, 2)),
            pltpu.SemaphoreType.DMA((DEPTH, 2)),
            pltpu.SemaphoreType.DMA((DEPTH, 2)),
            pltpu.SemaphoreType.DMA((DEPTH, 2)),
            pltpu.SemaphoreType.REGULAR,
            pltpu.SemaphoreType.REGULAR,
            pltpu.SemaphoreType.REGULAR,
            pltpu.SemaphoreType.REGULAR,
            pltpu.VMEM((5, 8, 128), jnp.float32),
            pltpu.VMEM((5, 8, 128), jnp.float32),
            pltpu.SemaphoreType.DMA((5,)),
            pltpu.SemaphoreType.DMA((5,)),
        ],
    )(x, w_mat, ring2mesh, mesh2ring)


# device time: 378074 ns/iter; 2.1095x vs baseline; 1.0316x over previous
import jax
import jax.numpy as jnp
from jax import lax
from jax.experimental import pallas as pl
from jax.experimental.pallas import tpu as pltpu

N_DEV = 32


def _mesh_idx(x, y, z):
    return z * 8 + y * 2 + (x if y % 2 == 0 else 1 - x)


def _ring_mesh_indices():
    p = []
    for z in range(4):
        ys = range(4) if z % 2 == 0 else range(3, -1, -1)
        for y in ys:
            p.append((y, z))
    cycle = [(0, y, z) for (y, z) in p] + [(1, y, z) for (y, z) in reversed(p)]
    return [_mesh_idx(x, y, z) for (x, y, z) in cycle]

_RING2MESH = _ring_mesh_indices()
_MESH2RING = [0] * N_DEV
for _r, _m in enumerate(_RING2MESH):
    _MESH2RING[_m] = _r

N_FWD = 16
N_BWD = 15
DEPTH = 4


def kernel(x, w_mat):
    m_per, k = x.shape
    _, n_per = w_mat.shape
    H = m_per // 2

    ring2mesh = jnp.array(_RING2MESH, dtype=jnp.int32)
    mesh2ring = jnp.array(_MESH2RING, dtype=jnp.int32)

    def body(x_ref, w_ref, r2m_ref, m2r_ref, out_ref,
             comm_f, comm_b, ssem_f, rsem_f, ssem_b, rsem_b,
             cred_f0, cred_f1, cred_b0, cred_b1,
             am_send, am_buf, am_ssem, am_rsem):
        my_pos = lax.axis_index("i")
        r_me = m2r_ref[my_pos]
        fwd_nbr = r2m_ref[lax.rem(r_me + 1, N_DEV)]
        bwd_nbr = r2m_ref[lax.rem(r_me - 1 + N_DEV, N_DEV)]
        cred_f = [cred_f0, cred_f1]
        cred_b = [cred_b0, cred_b1]

        barrier_sem = pltpu.get_barrier_semaphore()
        for nbr in (fwd_nbr, bwd_nbr):
            pl.semaphore_signal(
                barrier_sem, inc=1,
                device_id=(nbr,), device_id_type=pl.DeviceIdType.MESH,
            )
        pl.semaphore_wait(barrier_sem, 2)

        def origin_at(ring_offset):
            return r2m_ref[lax.rem(r_me + ring_offset + 2 * N_DEV, N_DEV)]

        def mk(comm, ssem, rsem, j, t, nbr):
            if j == 0:
                src = x_ref.at[t * H:(t + 1) * H]
            else:
                src = comm.at[j % DEPTH, t * H:(t + 1) * H]
            return pltpu.make_async_remote_copy(
                src_ref=src,
                dst_ref=comm.at[(j + 1) % DEPTH, t * H:(t + 1) * H],
                send_sem=ssem.at[j % DEPTH, t],
                recv_sem=rsem.at[(j + 1) % DEPTH, t],
                device_id=(nbr,),
                device_id_type=pl.DeviceIdType.MESH,
            )

        def half_gemm(src, origin, t, amax):
            res = jnp.dot(src, w_ref[...], preferred_element_type=jnp.float32)
            out_ref[pl.ds(origin * m_per + t * H, H), :] = res
            return jnp.maximum(amax, jnp.max(jnp.abs(res)))

        for t in (0, 1):
            mk(comm_f, ssem_f, rsem_f, 0, t, fwd_nbr).start()
        for t in (0, 1):
            mk(comm_b, ssem_b, rsem_b, 0, t, bwd_nbr).start()

        amax = jnp.float32(0.0)
        for t in (0, 1):
            amax = half_gemm(x_ref[t * H:(t + 1) * H], my_pos, t, amax)

        for s in range(1, N_FWD + 1):
            for t in (0, 1):
                mk(comm_f, ssem_f, rsem_f, s - 1, t, fwd_nbr).wait_recv()
                if s < N_FWD:
                    if s >= DEPTH - 1:
                        pl.semaphore_wait(cred_f[t], 1)
                    mk(comm_f, ssem_f, rsem_f, s, t, fwd_nbr).start()
            if s <= N_BWD:
                for t in (0, 1):
                    mk(comm_b, ssem_b, rsem_b, s - 1, t, bwd_nbr).wait_recv()
                    if s < N_BWD:
                        if s >= DEPTH - 1:
                            pl.semaphore_wait(cred_b[t], 1)
                        mk(comm_b, ssem_b, rsem_b, s, t, bwd_nbr).start()

            o_f = origin_at(-s)
            for t in (0, 1):
                amax = half_gemm(comm_f[s % DEPTH, t * H:(t + 1) * H], o_f, t, amax)
            if s <= N_BWD:
                o_b = origin_at(s)
                for t in (0, 1):
                    amax = half_gemm(comm_b[s % DEPTH, t * H:(t + 1) * H], o_b, t, amax)

            kk = s - 2
            if 0 <= kk <= N_FWD - 1:
                for t in (0, 1):
                    mk(comm_f, ssem_f, rsem_f, kk, t, fwd_nbr).wait_send()
                    if kk <= N_FWD - DEPTH:
                        pl.semaphore_signal(
                            cred_f[t], inc=1,
                            device_id=(bwd_nbr,),
                            device_id_type=pl.DeviceIdType.MESH,
                        )
            if 0 <= kk <= N_BWD - 1:
                for t in (0, 1):
                    mk(comm_b, ssem_b, rsem_b, kk, t, bwd_nbr).wait_send()
                    if kk <= N_BWD - DEPTH:
                        pl.semaphore_signal(
                            cred_b[t], inc=1,
                            device_id=(fwd_nbr,),
                            device_id_type=pl.DeviceIdType.MESH,
                        )

        for t in (0, 1):
            mk(comm_f, ssem_f, rsem_f, N_FWD - 1, t, fwd_nbr).wait_send()

        am_send[0] = jnp.full((8, 128), amax, dtype=jnp.float32)

        for j in range(N_DEV):
            @pl.when(j != my_pos)
            def _():
                rdma = pltpu.make_async_remote_copy(
                    src_ref=am_send.at[0],
                    dst_ref=am_buf.at[my_pos],
                    send_sem=am_ssem.at[j],
                    recv_sem=am_rsem.at[my_pos],
                    device_id=(j,),
                    device_id_type=pl.DeviceIdType.MESH,
                )
                rdma.start()
        am_buf[pl.ds(my_pos, 1)] = am_send[...]
        for j in range(N_DEV):
            @pl.when(j != my_pos)
            def _():
                recv = pltpu.make_async_remote_copy(
                    src_ref=am_send.at[0],
                    dst_ref=am_buf.at[j],
                    send_sem=am_ssem.at[j],
                    recv_sem=am_rsem.at[j],
                    device_id=(j,),
                    device_id_type=pl.DeviceIdType.MESH,
                )
                recv.wait_recv()
                recv.wait_send()
        amax = jnp.max(am_buf[...])

        scale = amax / 448.0
        q = jnp.clip(out_ref[...] / scale, -448.0, 448.0)
        out_ref[...] = q.astype(jnp.float8_e4m3fn).astype(jnp.float32) * scale

    return pl.pallas_call(
        body,
        out_shape=jax.ShapeDtypeStruct((N_DEV * m_per, n_per), jnp.float32),
        in_specs=[
            pl.BlockSpec(memory_space=pltpu.VMEM),
            pl.BlockSpec(memory_space=pltpu.VMEM),
            pl.BlockSpec(memory_space=pltpu.SMEM),
            pl.BlockSpec(memory_space=pltpu.SMEM),
        ],
        out_specs=pl.BlockSpec(memory_space=pltpu.VMEM),
        scratch_shapes=[
            pltpu.VMEM((DEPTH, m_per, k), jnp.float32),
            pltpu.VMEM((DEPTH, m_per, k), jnp.float32),
            pltpu.SemaphoreType.DMA((DEPTH, 2)),
            pltpu.SemaphoreType.DMA((DEPTH, 2)),
            pltpu.SemaphoreType.DMA((DEPTH, 2)),
            pltpu.SemaphoreType.DMA((DEPTH, 2)),
            pltpu.SemaphoreType.REGULAR,
            pltpu.SemaphoreType.REGULAR,
            pltpu.SemaphoreType.REGULAR,
            pltpu.SemaphoreType.REGULAR,
            pltpu.VMEM((1, 8, 128), jnp.float32),
            pltpu.VMEM((N_DEV, 8, 128), jnp.float32),
            pltpu.SemaphoreType.DMA((N_DEV,)),
            pltpu.SemaphoreType.DMA((N_DEV,)),
        ],
        compiler_params=pltpu.CompilerParams(collective_id=0),
    )(x, w_mat, ring2mesh, mesh2ring)
